# Initial kernel scaffold; baseline (speedup 1.0000x reference)
import jax
import jax.numpy as jnp
from jax.experimental import pallas as pl


def kernel(z, visited_seq):
    raise NotImplementedError("write your pallas kernel here")


# speedup vs baseline: 20.1822x; 20.1822x over previous
import functools

import jax
import jax.numpy as jnp
from jax import lax
from jax.experimental import pallas as pl
from jax.experimental.pallas import tpu as pltpu
from jax.experimental.pallas import tpu_sc as plsc

NUM_CORES = 2
NUM_SUBCORES = 16
NUM_TILES = NUM_CORES * NUM_SUBCORES

CHUNK_B = 80
LANES = 16


def _make_hist(nbins: int, chunks: int):
    mesh = plsc.VectorSubcoreMesh(core_axis_name="c", subcore_axis_name="s")

    @functools.partial(
        pl.kernel,
        mesh=mesh,
        out_type=jax.ShapeDtypeStruct((NUM_CORES, nbins), jnp.float32),
        scratch_types=[
            pltpu.VMEM((chunks, CHUNK_B), jnp.int32),
            pltpu.VMEM((CHUNK_B,), jnp.float32),
            pltpu.VMEM((nbins,), jnp.float32),
            pltpu.VMEM_SHARED((nbins,), jnp.float32),
        ],
    )
    def hist(idx_hbm, out_hbm, idx_v, ones_v, tmp_v, counts_sh):
        c = lax.axis_index("c")
        s = lax.axis_index("s")
        wid = s * NUM_CORES + c

        pltpu.sync_copy(idx_hbm.at[wid], idx_v)

        for i in range(CHUNK_B // LANES):
            ones_v[pl.ds(i * LANES, LANES)] = jnp.ones((LANES,), jnp.float32)

        @pl.when(s == 0)
        def _zero():
            def zbody(i, carry):
                tmp_v[pl.ds(i * LANES, LANES)] = jnp.zeros((LANES,), jnp.float32)
                return carry
            lax.fori_loop(0, nbins // LANES, zbody, 0)
            pltpu.sync_copy(tmp_v, counts_sh)

        plsc.subcore_barrier()

        def body(j, carry):
            pltpu.sync_copy(ones_v, counts_sh.at[idx_v.at[j]], add=True)
            return carry

        lax.fori_loop(0, chunks, body, 0)

        plsc.subcore_barrier()

        @pl.when(s == 0)
        def _emit():
            pltpu.sync_copy(counts_sh, out_hbm.at[c])

    return hist


def _matvec_body(scale, p_ref, z_ref, o_ref):
    counts = (p_ref[0:1, :] + p_ref[1:2, :]) * scale
    o_ref[...] = lax.dot_general(
        counts, z_ref[...], (((1,), (0,)), ((), ())),
        preferred_element_type=jnp.float32,
        precision=lax.Precision.HIGHEST,
    )


def kernel(z, visited_seq):
    nbins, d = z.shape
    n = visited_seq.shape[0]
    assert n % NUM_TILES == 0
    per_tile = n // NUM_TILES
    assert per_tile % CHUNK_B == 0
    chunks = per_tile // CHUNK_B

    idx = visited_seq.astype(jnp.int32).reshape(NUM_TILES, chunks, CHUNK_B)
    partials = _make_hist(nbins, chunks)(idx)

    out = pl.pallas_call(
        functools.partial(_matvec_body, 1.0 / n),
        out_shape=jax.ShapeDtypeStruct((1, d), jnp.float32),
    )(partials, z)
    return out


# speedup vs baseline: 26.2253x; 1.2994x over previous
import functools

import jax
import jax.numpy as jnp
from jax import lax
from jax.experimental import pallas as pl
from jax.experimental.pallas import tpu as pltpu
from jax.experimental.pallas import tpu_sc as plsc

NUM_CORES = 2
NUM_SUBCORES = 16
NUM_TILES = NUM_CORES * NUM_SUBCORES

CHUNK_B = 80
GROUP = 5
LANES = 16


def _make_hist(nbins: int, per_tile: int):
    chunks = per_tile // CHUNK_B
    assert chunks % GROUP == 0
    mesh = plsc.VectorSubcoreMesh(core_axis_name="c", subcore_axis_name="s")

    @functools.partial(
        pl.kernel,
        mesh=mesh,
        out_type=jax.ShapeDtypeStruct((NUM_CORES, nbins), jnp.float32),
        scratch_types=[
            pltpu.VMEM((per_tile,), jnp.int32),
            pltpu.VMEM((CHUNK_B,), jnp.float32),
            pltpu.VMEM((nbins,), jnp.float32),
            pltpu.VMEM_SHARED((nbins,), jnp.float32),
            pltpu.SemaphoreType.DMA,
        ],
    )
    def hist(idx_hbm, out_hbm, idx_v, ones_v, tmp_v, counts_sh, sem):
        c = lax.axis_index("c")
        s = lax.axis_index("s")
        wid = s * NUM_CORES + c

        pltpu.sync_copy(idx_hbm.at[pl.ds(wid * per_tile, per_tile)], idx_v)

        for i in range(CHUNK_B // LANES):
            ones_v[pl.ds(i * LANES, LANES)] = jnp.ones((LANES,), jnp.float32)

        @pl.when(s == 0)
        def _zero():
            def zbody(i, carry):
                tmp_v[pl.ds(i * LANES, LANES)] = jnp.zeros((LANES,), jnp.float32)
                return carry
            lax.fori_loop(0, nbins // LANES, zbody, 0)
            pltpu.sync_copy(tmp_v, counts_sh)

        plsc.subcore_barrier()

        def body(g, carry):
            descs = []
            for b in range(GROUP):
                off = (g * GROUP + b) * CHUNK_B
                descs.append(pltpu.async_copy(
                    ones_v, counts_sh.at[idx_v.at[pl.ds(off, CHUNK_B)]], sem,
                    add=True))
            for d in descs:
                d.wait()
            return carry

        lax.fori_loop(0, chunks // GROUP, body, 0)

        plsc.subcore_barrier()

        @pl.when(s == 0)
        def _emit():
            pltpu.sync_copy(counts_sh, out_hbm.at[c])

    return hist


def _matvec_body(scale, p_ref, z_ref, o_ref):
    counts = (p_ref[0:1, :] + p_ref[1:2, :]) * scale
    o_ref[...] = lax.dot_general(
        counts, z_ref[...], (((1,), (0,)), ((), ())),
        preferred_element_type=jnp.float32,
        precision=lax.Precision.DEFAULT,
    )


def kernel(z, visited_seq):
    nbins, d = z.shape
    n = visited_seq.shape[0]
    assert n % NUM_TILES == 0
    per_tile = n // NUM_TILES
    assert per_tile % CHUNK_B == 0

    idx = visited_seq.astype(jnp.int32)
    partials = _make_hist(nbins, per_tile)(idx)

    out = pl.pallas_call(
        functools.partial(_matvec_body, 1.0 / n),
        out_shape=jax.ShapeDtypeStruct((1, d), jnp.float32),
    )(partials, z)
    return out


# speedup vs baseline: 27.1191x; 1.0341x over previous
import functools

import jax
import jax.numpy as jnp
from jax import lax
from jax.experimental import pallas as pl
from jax.experimental.pallas import tpu as pltpu
from jax.experimental.pallas import tpu_sc as plsc

NUM_CORES = 2
NUM_SUBCORES = 16
NUM_TILES = NUM_CORES * NUM_SUBCORES
LANES = 16


def _make_hist(nbins: int, per_tile: int):
    mesh = plsc.VectorSubcoreMesh(core_axis_name="c", subcore_axis_name="s")

    @functools.partial(
        pl.kernel,
        mesh=mesh,
        out_type=jax.ShapeDtypeStruct((NUM_TILES, nbins), jnp.float32),
        scratch_types=[
            pltpu.VMEM((per_tile,), jnp.int32),
            pltpu.VMEM((nbins,), jnp.float32),
        ],
        compiler_params=pltpu.CompilerParams(needs_layout_passes=False),
    )
    def hist(idx_hbm, out_hbm, idx_v, counts_v):
        c = lax.axis_index("c")
        s = lax.axis_index("s")
        wid = s * NUM_CORES + c

        pltpu.sync_copy(idx_hbm.at[pl.ds(wid * per_tile, per_tile)], idx_v)

        def zbody(i, carry):
            counts_v[pl.ds(i * LANES, LANES)] = jnp.zeros((LANES,), jnp.float32)
            return carry
        lax.fori_loop(0, nbins // LANES, zbody, 0)

        def body(i, carry):
            idx16 = idx_v[pl.ds(i * LANES, LANES)]
            plsc.addupdate_scatter(
                counts_v, [idx16], jnp.ones((LANES,), jnp.float32))
            return carry

        lax.fori_loop(0, per_tile // LANES, body, 0)

        pltpu.sync_copy(counts_v, out_hbm.at[wid])

    return hist


def _matvec_body(scale, p_ref, z_ref, o_ref):
    counts = jnp.sum(p_ref[...], axis=0, keepdims=True) * scale
    o_ref[...] = lax.dot_general(
        counts, z_ref[...], (((1,), (0,)), ((), ())),
        preferred_element_type=jnp.float32,
        precision=lax.Precision.DEFAULT,
    )


def kernel(z, visited_seq):
    nbins, d = z.shape
    n = visited_seq.shape[0]
    assert n % (NUM_TILES * LANES) == 0
    per_tile = n // NUM_TILES

    idx = visited_seq.astype(jnp.int32)
    partials = _make_hist(nbins, per_tile)(idx)

    out = pl.pallas_call(
        functools.partial(_matvec_body, 1.0 / n),
        out_shape=jax.ShapeDtypeStruct((1, d), jnp.float32),
    )(partials, z)
    return out


# speedup vs baseline: 29.1120x; 1.0735x over previous
import functools

import jax
import jax.numpy as jnp
from jax import lax
from jax.experimental import pallas as pl
from jax.experimental.pallas import tpu as pltpu
from jax.experimental.pallas import tpu_sc as plsc

NUM_CORES = 2
NUM_SUBCORES = 16
NUM_TILES = NUM_CORES * NUM_SUBCORES
LANES = 16


def _make_hist(nbins: int, per_tile: int):
    mesh = plsc.VectorSubcoreMesh(core_axis_name="c", subcore_axis_name="s")

    @functools.partial(
        pl.kernel,
        mesh=mesh,
        out_type=jax.ShapeDtypeStruct((NUM_TILES, nbins), jnp.float32),
        scratch_types=[
            pltpu.VMEM((per_tile,), jnp.int32),
            pltpu.VMEM((nbins,), jnp.float32),
        ],
        compiler_params=pltpu.CompilerParams(needs_layout_passes=False),
    )
    def hist(idx_hbm, out_hbm, idx_v, counts_v):
        c = lax.axis_index("c")
        s = lax.axis_index("s")
        wid = s * NUM_CORES + c

        pltpu.sync_copy(idx_hbm.at[pl.ds(wid * per_tile, per_tile)], idx_v)

        zu = 25
        assert nbins % (LANES * zu) == 0

        def zbody(i, carry):
            for u in range(zu):
                counts_v[pl.ds((i * zu + u) * LANES, LANES)] = (
                    jnp.zeros((LANES,), jnp.float32))
            return carry
        lax.fori_loop(0, nbins // (LANES * zu), zbody, 0)

        su = 25
        assert per_tile % (LANES * su) == 0

        def body(i, carry):
            for u in range(su):
                idx16 = idx_v[pl.ds((i * su + u) * LANES, LANES)]
                plsc.addupdate_scatter(
                    counts_v, [idx16], jnp.ones((LANES,), jnp.float32))
            return carry

        lax.fori_loop(0, per_tile // (LANES * su), body, 0)

        pltpu.sync_copy(counts_v, out_hbm.at[wid])

    return hist


def _matvec_body(scale, p_ref, z_ref, o_ref):
    counts = jnp.sum(p_ref[...], axis=0, keepdims=True) * scale
    o_ref[...] = lax.dot_general(
        counts, z_ref[...], (((1,), (0,)), ((), ())),
        preferred_element_type=jnp.float32,
        precision=lax.Precision.DEFAULT,
    )


def kernel(z, visited_seq):
    nbins, d = z.shape
    n = visited_seq.shape[0]
    assert n % (NUM_TILES * LANES) == 0
    per_tile = n // NUM_TILES

    idx = visited_seq.astype(jnp.int32)
    partials = _make_hist(nbins, per_tile)(idx)

    out = pl.pallas_call(
        functools.partial(_matvec_body, 1.0 / n),
        out_shape=jax.ShapeDtypeStruct((1, d), jnp.float32),
    )(partials, z)
    return out
